# Initial kernel scaffold; baseline (speedup 1.0000x reference)
import jax
import jax.numpy as jnp
from jax.experimental import pallas as pl


def kernel(x, temperature, w_qkv, b_qkv, w_dw, b_dw, w_proj, b_proj):
    raise NotImplementedError("write your pallas kernel here")


# speedup vs baseline: 1.8590x; 1.8590x over previous
import functools

import jax
import jax.numpy as jnp
from jax import lax
from jax.experimental import pallas as pl
from jax.experimental.pallas import tpu as pltpu


def _fused_attention_kernel(x_ref, wqkv_ref, bqkv_ref, wdw_ref, bdw_ref,
                            wproj_ref, bproj_ref, temp_ref, o_ref,
                            *, H, W, num_heads):
    f32 = jnp.float32
    C = x_ref.shape[1]
    HW = H * W
    x = x_ref[0]

    qkv = jnp.dot(wqkv_ref[...], x, preferred_element_type=f32)
    qkv = qkv + bqkv_ref[...]

    pos = lax.broadcasted_iota(jnp.int32, (1, HW), 1)
    jcol = pos % W
    yrow = pos // W

    x_lft = jnp.where(jcol == 0, 0.0, pltpu.roll(qkv, 1, axis=1))
    x_rgt = jnp.where(jcol == W - 1, 0.0, pltpu.roll(qkv, HW - 1, axis=1))

    wdw = wdw_ref[...]

    def tap(i):
        return wdw[:, i:i + 1]

    a_top = x_lft * tap(0) + qkv * tap(1) + x_rgt * tap(2)
    a_mid = x_lft * tap(3) + qkv * tap(4) + x_rgt * tap(5)
    a_bot = x_lft * tap(6) + qkv * tap(7) + x_rgt * tap(8)

    g = a_mid
    g = g + jnp.where(yrow == 0, 0.0, pltpu.roll(a_top, W, axis=1))
    g = g + jnp.where(yrow == H - 1, 0.0, pltpu.roll(a_bot, HW - W, axis=1))
    g = g + bdw_ref[...]

    q = g[0:C]
    k = g[C:2 * C]
    v = g[2 * C:3 * C]

    ones_row = jnp.ones((1, HW), f32)
    sq = lax.dot_general(q * q, ones_row, (((1,), (1,)), ((), ())),
                         preferred_element_type=f32)
    sk = lax.dot_general(ones_row, k * k, (((1,), (1,)), ((), ())),
                         preferred_element_type=f32)
    gram = lax.dot_general(q, k, (((1,), (1,)), ((), ())),
                           preferred_element_type=f32)

    eps2 = 1e-24
    qn = lax.rsqrt(jnp.maximum(sq, eps2))
    kn = lax.rsqrt(jnp.maximum(sk, eps2))
    s = gram * qn * kn * temp_ref[...]

    c = C // num_heads
    ri = lax.broadcasted_iota(jnp.int32, (C, C), 0)
    ci = lax.broadcasted_iota(jnp.int32, (C, C), 1)
    s = jnp.where((ri // c) == (ci // c), s, -1e30)
    s = s - jnp.max(s, axis=-1, keepdims=True)
    e = jnp.exp(s)
    p = e / jnp.sum(e, axis=-1, keepdims=True)

    m = lax.dot_general(p, wproj_ref[...], (((0,), (0,)), ((), ())),
                        preferred_element_type=f32)
    out = lax.dot_general(m, v, (((0,), (0,)), ((), ())),
                          preferred_element_type=f32)
    o_ref[0] = out + bproj_ref[...]


def kernel(x, temperature, w_qkv, b_qkv, w_dw, b_dw, w_proj, b_proj):
    B, C, H, W = x.shape
    HW = H * W
    num_heads = temperature.shape[0]
    C3 = 3 * C

    x_cm = x.reshape(B, C, HW)
    wqkv_t = w_qkv.T
    bqkv_c = b_qkv.reshape(C3, 1)
    wdw_t = w_dw.reshape(9, C3).T
    bdw_c = b_dw.reshape(C3, 1)
    bproj_c = b_proj.reshape(C, 1)
    temp_col = jnp.repeat(temperature.astype(jnp.float32),
                          C // num_heads).reshape(C, 1)

    body = functools.partial(_fused_attention_kernel,
                             H=H, W=W, num_heads=num_heads)
    out = pl.pallas_call(
        body,
        out_shape=jax.ShapeDtypeStruct((B, C, HW), jnp.float32),
        grid=(B,),
        in_specs=[
            pl.BlockSpec((1, C, HW), lambda b: (b, 0, 0)),
            pl.BlockSpec((C3, C), lambda b: (0, 0)),
            pl.BlockSpec((C3, 1), lambda b: (0, 0)),
            pl.BlockSpec((C3, 9), lambda b: (0, 0)),
            pl.BlockSpec((C3, 1), lambda b: (0, 0)),
            pl.BlockSpec((C, C), lambda b: (0, 0)),
            pl.BlockSpec((C, 1), lambda b: (0, 0)),
            pl.BlockSpec((C, 1), lambda b: (0, 0)),
        ],
        out_specs=pl.BlockSpec((1, C, HW), lambda b: (b, 0, 0)),
        compiler_params=pltpu.CompilerParams(
            dimension_semantics=("parallel",),
            vmem_limit_bytes=64 * 1024 * 1024,
        ),
    )(x_cm, wqkv_t, bqkv_c, wdw_t, bdw_c, w_proj, bproj_c, temp_col)
    return out.reshape(B, C, H, W)


# speedup vs baseline: 2.3651x; 1.2722x over previous
import functools

import jax
import jax.numpy as jnp
from jax import lax
from jax.experimental import pallas as pl
from jax.experimental.pallas import tpu as pltpu


def _fused_attention_kernel(x_ref, w3_ref, wproj_ref, bproj_ref, temp_ref,
                            o_ref, *, H, W, num_heads):
    f32 = jnp.float32
    bf16 = jnp.bfloat16
    C = x_ref.shape[1]
    HW = H * W

    pos = lax.broadcasted_iota(jnp.int32, (1, HW), 1)
    jcol = pos % W
    yrow = pos // W

    cl = jnp.where(jcol == 0, 0.0, 1.0).astype(bf16)
    cr = jnp.where(jcol == W - 1, 0.0, 1.0).astype(bf16)
    rt = jnp.where(yrow == 0, 0.0, 1.0).astype(bf16)
    rb = jnp.where(yrow == H - 1, 0.0, 1.0).astype(bf16)

    xb = x_ref[0].astype(bf16)
    xl = pltpu.roll(xb, 1, axis=1) * cl
    xr = pltpu.roll(xb, HW - 1, axis=1) * cr

    def down(a):
        return pltpu.roll(a, W, axis=1) * rt

    def up(a):
        return pltpu.roll(a, HW - W, axis=1) * rb

    one = jnp.ones((1, HW), bf16)
    zrow = jnp.zeros((6, HW), bf16)

    xcat = jnp.concatenate([
        down(xl), down(xb), down(xr),
        xl, xb, xr,
        up(xl), up(xb), up(xr),
        cl * rt, rt, cr * rt,
        cl, one, cr,
        cl * rb, rb, cr * rb,
        one, zrow,
    ], axis=0)

    g = jnp.dot(w3_ref[...], xcat, preferred_element_type=f32)

    q = g[0:C]
    k = g[C:2 * C]
    v = g[2 * C:3 * C]

    ones_row = jnp.ones((1, HW), f32)
    sq = lax.dot_general(q * q, ones_row, (((1,), (1,)), ((), ())),
                         preferred_element_type=f32)
    sk = lax.dot_general(ones_row, k * k, (((1,), (1,)), ((), ())),
                         preferred_element_type=f32)
    gram = lax.dot_general(q, k, (((1,), (1,)), ((), ())),
                           preferred_element_type=f32)

    eps2 = 1e-24
    qn = lax.rsqrt(jnp.maximum(sq, eps2))
    kn = lax.rsqrt(jnp.maximum(sk, eps2))
    s = gram * qn * kn * temp_ref[...]

    c = C // num_heads
    ri = lax.broadcasted_iota(jnp.int32, (C, C), 0)
    ci = lax.broadcasted_iota(jnp.int32, (C, C), 1)
    s = jnp.where((ri // c) == (ci // c), s, -1e30)
    s = s - jnp.max(s, axis=-1, keepdims=True)
    e = jnp.exp(s)
    p = e / jnp.sum(e, axis=-1, keepdims=True)

    m = lax.dot_general(p, wproj_ref[...], (((0,), (0,)), ((), ())),
                        preferred_element_type=f32)
    out = lax.dot_general(m, v, (((0,), (0,)), ((), ())),
                          preferred_element_type=f32)
    o_ref[0] = out + bproj_ref[...]


def kernel(x, temperature, w_qkv, b_qkv, w_dw, b_dw, w_proj, b_proj):
    B, C, H, W = x.shape
    HW = H * W
    num_heads = temperature.shape[0]
    C3 = 3 * C

    x_cm = x.reshape(B, C, HW)

    wdw9 = w_dw.reshape(9, C3)
    w3 = jnp.einsum('ec,tc->cte', w_qkv, wdw9)
    w3 = w3.reshape(C3, 9 * C)
    wb9 = (b_qkv[None, :] * wdw9).T
    w3_full = jnp.concatenate(
        [w3, wb9, b_dw.reshape(C3, 1), jnp.zeros((C3, 6), w3.dtype)],
        axis=1).astype(jnp.bfloat16)

    bproj_c = b_proj.reshape(C, 1)
    temp_col = jnp.repeat(temperature.astype(jnp.float32),
                          C // num_heads).reshape(C, 1)

    body = functools.partial(_fused_attention_kernel,
                             H=H, W=W, num_heads=num_heads)
    K = 9 * C + 16
    out = pl.pallas_call(
        body,
        out_shape=jax.ShapeDtypeStruct((B, C, HW), jnp.float32),
        grid=(B,),
        in_specs=[
            pl.BlockSpec((1, C, HW), lambda b: (b, 0, 0)),
            pl.BlockSpec((C3, K), lambda b: (0, 0)),
            pl.BlockSpec((C, C), lambda b: (0, 0)),
            pl.BlockSpec((C, 1), lambda b: (0, 0)),
            pl.BlockSpec((C, 1), lambda b: (0, 0)),
        ],
        out_specs=pl.BlockSpec((1, C, HW), lambda b: (b, 0, 0)),
        compiler_params=pltpu.CompilerParams(
            dimension_semantics=("parallel",),
            vmem_limit_bytes=64 * 1024 * 1024,
        ),
    )(x_cm, w3_full, w_proj, bproj_c, temp_col)
    return out.reshape(B, C, H, W)


# speedup vs baseline: 2.4019x; 1.0155x over previous
import functools

import jax
import jax.numpy as jnp
from jax import lax
from jax.experimental import pallas as pl
from jax.experimental.pallas import tpu as pltpu


def _fused_attention_kernel(x_ref, w3_ref, wproj_ref, bproj_ref, temp_ref,
                            o_ref, *, H, W, num_heads):
    f32 = jnp.float32
    bf16 = jnp.bfloat16
    C = x_ref.shape[1]
    HW = H * W
    nb = x_ref.shape[0]

    pos = lax.broadcasted_iota(jnp.int32, (1, HW), 1)
    jcol = pos % W
    yrow = pos // W

    cl = jnp.where(jcol == 0, 0.0, 1.0).astype(bf16)
    cr = jnp.where(jcol == W - 1, 0.0, 1.0).astype(bf16)
    rt = jnp.where(yrow == 0, 0.0, 1.0).astype(bf16)
    rb = jnp.where(yrow == H - 1, 0.0, 1.0).astype(bf16)

    one = jnp.ones((1, HW), bf16)
    zrow = jnp.zeros((6, HW), bf16)
    ind = [cl * rt, rt, cr * rt,
           cl, one, cr,
           cl * rb, rb, cr * rb,
           one, zrow]

    c = C // num_heads
    ri = lax.broadcasted_iota(jnp.int32, (C, C), 0)
    ci = lax.broadcasted_iota(jnp.int32, (C, C), 1)
    same_head = (ri // c) == (ci // c)
    ones_row = jnp.ones((1, HW), f32)
    eps2 = 1e-24

    for i in range(nb):
        xb = x_ref[i].astype(bf16)
        xl = pltpu.roll(xb, 1, axis=1) * cl
        xr = pltpu.roll(xb, HW - 1, axis=1) * cr

        def down(a):
            return pltpu.roll(a, W, axis=1) * rt

        def up(a):
            return pltpu.roll(a, HW - W, axis=1) * rb

        xcat = jnp.concatenate([
            down(xl), down(xb), down(xr),
            xl, xb, xr,
            up(xl), up(xb), up(xr),
        ] + ind, axis=0)

        g = jnp.dot(w3_ref[...], xcat,
                    preferred_element_type=f32)

        q = g[0:C]
        k = g[C:2 * C]
        v = g[2 * C:3 * C]

        sq = lax.dot_general(q * q, ones_row, (((1,), (1,)), ((), ())),
                             preferred_element_type=f32)
        sk = lax.dot_general(ones_row, k * k, (((1,), (1,)), ((), ())),
                             preferred_element_type=f32)
        gram = lax.dot_general(q, k, (((1,), (1,)), ((), ())),
                               preferred_element_type=f32)

        qn = lax.rsqrt(jnp.maximum(sq, eps2))
        kn = lax.rsqrt(jnp.maximum(sk, eps2))
        s = gram * qn * kn * temp_ref[...]

        s = jnp.where(same_head, s, -1e30)
        s = s - jnp.max(s, axis=-1, keepdims=True)
        e = jnp.exp(s)
        p = e / jnp.sum(e, axis=-1, keepdims=True)

        m = lax.dot_general(p, wproj_ref[...], (((0,), (0,)), ((), ())),
                            preferred_element_type=f32)
        out = lax.dot_general(m, v, (((0,), (0,)), ((), ())),
                              preferred_element_type=f32)
        o_ref[i] = out + bproj_ref[...]


def kernel(x, temperature, w_qkv, b_qkv, w_dw, b_dw, w_proj, b_proj):
    B, C, H, W = x.shape
    HW = H * W
    num_heads = temperature.shape[0]
    C3 = 3 * C

    x_cm = x.reshape(B, C, HW)

    wdw9 = w_dw.reshape(9, C3)
    w3 = jnp.einsum('ec,tc->cte', w_qkv, wdw9)
    w3 = w3.reshape(C3, 9 * C)
    wb9 = (b_qkv[None, :] * wdw9).T
    w3_full = jnp.concatenate(
        [w3, wb9, b_dw.reshape(C3, 1), jnp.zeros((C3, 6), w3.dtype)],
        axis=1).astype(jnp.bfloat16)

    bproj_c = b_proj.reshape(C, 1)
    temp_col = jnp.repeat(temperature.astype(jnp.float32),
                          C // num_heads).reshape(C, 1)

    body = functools.partial(_fused_attention_kernel,
                             H=H, W=W, num_heads=num_heads)
    K = 9 * C + 16
    NB = 2
    out = pl.pallas_call(
        body,
        out_shape=jax.ShapeDtypeStruct((B, C, HW), jnp.float32),
        grid=(B // NB,),
        in_specs=[
            pl.BlockSpec((NB, C, HW), lambda b: (b, 0, 0)),
            pl.BlockSpec((C3, K), lambda b: (0, 0)),
            pl.BlockSpec((C, C), lambda b: (0, 0)),
            pl.BlockSpec((C, 1), lambda b: (0, 0)),
            pl.BlockSpec((C, 1), lambda b: (0, 0)),
        ],
        out_specs=pl.BlockSpec((NB, C, HW), lambda b: (b, 0, 0)),
        compiler_params=pltpu.CompilerParams(
            dimension_semantics=("parallel",),
            vmem_limit_bytes=64 * 1024 * 1024,
        ),
    )(x_cm, w3_full, w_proj, bproj_c, temp_col)
    return out.reshape(B, C, H, W)


# speedup vs baseline: 3.8204x; 1.5906x over previous
import functools

import jax
import jax.numpy as jnp
from jax import lax
from jax.experimental import pallas as pl
from jax.experimental.pallas import tpu as pltpu


def _fused_attention_kernel(x_ref, w3_ref, wproj_ref, bproj_ref, temp_ref,
                            o_ref, *, H, W, num_heads):
    f32 = jnp.float32
    bf16 = jnp.bfloat16
    C = x_ref.shape[1]
    HW = H * W
    nb = x_ref.shape[0]

    pos = lax.broadcasted_iota(jnp.int32, (1, HW), 1)
    jcol = pos % W
    yrow = pos // W

    cl = jnp.where(jcol == 0, 0.0, 1.0).astype(bf16)
    cr = jnp.where(jcol == W - 1, 0.0, 1.0).astype(bf16)
    rt = jnp.where(yrow == 0, 0.0, 1.0).astype(bf16)
    rb = jnp.where(yrow == H - 1, 0.0, 1.0).astype(bf16)

    one = jnp.ones((1, HW), bf16)
    zrow = jnp.zeros((6, HW), bf16)
    ind = [cl * rt, rt, cr * rt,
           cl, one, cr,
           cl * rb, rb, cr * rb,
           one, zrow]

    c = C // num_heads
    ri = lax.broadcasted_iota(jnp.int32, (C, C), 0)
    ci = lax.broadcasted_iota(jnp.int32, (C, C), 1)
    same_head = (ri // c) == (ci // c)
    ones_row = jnp.ones((1, HW), f32)
    eps2 = 1e-24

    for i in range(nb):
        xb = x_ref[i].reshape(C, HW).astype(bf16)
        xl = pltpu.roll(xb, 1, axis=1) * cl
        xr = pltpu.roll(xb, HW - 1, axis=1) * cr

        def down(a):
            return pltpu.roll(a, W, axis=1) * rt

        def up(a):
            return pltpu.roll(a, HW - W, axis=1) * rb

        xcat = jnp.concatenate([
            down(xl), down(xb), down(xr),
            xl, xb, xr,
            up(xl), up(xb), up(xr),
        ] + ind, axis=0)

        g = jnp.dot(w3_ref[...], xcat,
                    preferred_element_type=f32)

        q = g[0:C]
        k = g[C:2 * C]
        v = g[2 * C:3 * C]

        sq = lax.dot_general(q * q, ones_row, (((1,), (1,)), ((), ())),
                             preferred_element_type=f32)
        sk = lax.dot_general(ones_row, k * k, (((1,), (1,)), ((), ())),
                             preferred_element_type=f32)
        gram = lax.dot_general(q, k, (((1,), (1,)), ((), ())),
                               preferred_element_type=f32)

        qn = lax.rsqrt(jnp.maximum(sq, eps2))
        kn = lax.rsqrt(jnp.maximum(sk, eps2))
        s = gram * qn * kn * temp_ref[...]

        s = jnp.where(same_head, s, -1e30)
        s = s - jnp.max(s, axis=-1, keepdims=True)
        e = jnp.exp(s)
        p = e / jnp.sum(e, axis=-1, keepdims=True)

        m = lax.dot_general(p, wproj_ref[...], (((0,), (0,)), ((), ())),
                            preferred_element_type=f32)
        out = lax.dot_general(m, v, (((0,), (0,)), ((), ())),
                              preferred_element_type=f32)
        o_ref[i] = (out + bproj_ref[...]).reshape(C, H, W)


def kernel(x, temperature, w_qkv, b_qkv, w_dw, b_dw, w_proj, b_proj):
    B, C, H, W = x.shape
    HW = H * W
    num_heads = temperature.shape[0]
    C3 = 3 * C

    wdw9 = w_dw.reshape(9, C3)
    w3 = jnp.einsum('ec,tc->cte', w_qkv, wdw9)
    w3 = w3.reshape(C3, 9 * C)
    wb9 = (b_qkv[None, :] * wdw9).T
    w3_full = jnp.concatenate(
        [w3, wb9, b_dw.reshape(C3, 1), jnp.zeros((C3, 6), w3.dtype)],
        axis=1).astype(jnp.bfloat16)

    bproj_c = b_proj.reshape(C, 1)
    temp_col = jnp.repeat(temperature.astype(jnp.float32),
                          C // num_heads).reshape(C, 1)

    body = functools.partial(_fused_attention_kernel,
                             H=H, W=W, num_heads=num_heads)
    K = 9 * C + 16
    NB = 2
    out = pl.pallas_call(
        body,
        out_shape=jax.ShapeDtypeStruct((B, C, H, W), jnp.float32),
        grid=(B // NB,),
        in_specs=[
            pl.BlockSpec((NB, C, H, W), lambda b: (b, 0, 0, 0)),
            pl.BlockSpec((C3, K), lambda b: (0, 0)),
            pl.BlockSpec((C, C), lambda b: (0, 0)),
            pl.BlockSpec((C, 1), lambda b: (0, 0)),
            pl.BlockSpec((C, 1), lambda b: (0, 0)),
        ],
        out_specs=pl.BlockSpec((NB, C, H, W), lambda b: (b, 0, 0, 0)),
        compiler_params=pltpu.CompilerParams(
            dimension_semantics=("parallel",),
            vmem_limit_bytes=64 * 1024 * 1024,
        ),
    )(x, w3_full, w_proj, bproj_c, temp_col)
    return out


# speedup vs baseline: 4.5858x; 1.2003x over previous
import functools

import jax
import jax.numpy as jnp
from jax import lax
from jax.experimental import pallas as pl
from jax.experimental.pallas import tpu as pltpu


def _build_xcat(x_ref, dst_ref, cl, cr, rt, rb, *, C, W, HW):
    bf16 = jnp.bfloat16
    xb = x_ref[0].astype(bf16).reshape(C, HW)
    xl = pltpu.roll(xb, 1, axis=1) * cl
    xr = pltpu.roll(xb, HW - 1, axis=1) * cr

    def down(a):
        return pltpu.roll(a, W, axis=1) * rt

    def up(a):
        return pltpu.roll(a, HW - W, axis=1) * rb

    for t, blk in enumerate([
            down(xl), down(xb), down(xr),
            xl, xb, xr,
            up(xl), up(xb), up(xr)]):
        dst_ref[t * C:(t + 1) * C, :] = blk


def _attend(src_ref, w3_ref, wproj_ref, bproj_ref, temp_ref, o_ref,
            *, C, H, W, HW, num_heads):
    f32 = jnp.float32
    g = jnp.dot(w3_ref[...], src_ref[...],
                preferred_element_type=f32)
    q = g[0:C]
    k = g[C:2 * C]
    v = g[2 * C:3 * C]

    ones_row = jnp.ones((1, HW), f32)
    sq = lax.dot_general(q * q, ones_row, (((1,), (1,)), ((), ())),
                         preferred_element_type=f32)
    sk = lax.dot_general(ones_row, k * k, (((1,), (1,)), ((), ())),
                         preferred_element_type=f32)
    gram = lax.dot_general(q, k, (((1,), (1,)), ((), ())),
                           preferred_element_type=f32)

    eps2 = 1e-24
    qn = lax.rsqrt(jnp.maximum(sq, eps2))
    kn = lax.rsqrt(jnp.maximum(sk, eps2))
    sc = gram * qn * kn * temp_ref[...]

    hc = C // num_heads
    ri = lax.broadcasted_iota(jnp.int32, (C, C), 0)
    ci = lax.broadcasted_iota(jnp.int32, (C, C), 1)
    sc = jnp.where((ri // hc) == (ci // hc), sc, -1e30)
    sc = sc - jnp.max(sc, axis=-1, keepdims=True)
    e = jnp.exp(sc)
    p = e / jnp.sum(e, axis=-1, keepdims=True)

    m = lax.dot_general(p, wproj_ref[...], (((0,), (0,)), ((), ())),
                        preferred_element_type=f32)
    out = lax.dot_general(m, v, (((0,), (0,)), ((), ())),
                          preferred_element_type=f32)
    o_ref[0] = (out + bproj_ref[...]).reshape(C, H, W)


def _fused_attention_kernel(x_ref, w3_ref, wproj_ref, bproj_ref, temp_ref,
                            o_ref, xcat0_ref, xcat1_ref,
                            *, H, W, num_heads):
    bf16 = jnp.bfloat16
    C = x_ref.shape[1]
    HW = H * W
    s = pl.program_id(0)

    pos = lax.broadcasted_iota(jnp.int32, (1, HW), 1)
    jcol = pos % W
    yrow = pos // W

    cl = jnp.where(jcol == 0, 0.0, 1.0).astype(bf16)
    cr = jnp.where(jcol == W - 1, 0.0, 1.0).astype(bf16)
    rt = jnp.where(yrow == 0, 0.0, 1.0).astype(bf16)
    rb = jnp.where(yrow == H - 1, 0.0, 1.0).astype(bf16)

    @pl.when(s == 0)
    def _():
        one = jnp.ones((1, HW), bf16)
        zrow = jnp.zeros((6, HW), bf16)
        indcat = jnp.concatenate([
            cl * rt, rt, cr * rt,
            cl, one, cr,
            cl * rb, rb, cr * rb,
            one, zrow,
        ], axis=0)
        xcat0_ref[9 * C:, :] = indcat
        xcat1_ref[9 * C:, :] = indcat
        xcat1_ref[0:9 * C, :] = jnp.zeros((9 * C, HW), bf16)

    bk = dict(C=C, W=W, HW=HW)
    ak = dict(C=C, H=H, W=W, HW=HW, num_heads=num_heads)

    @pl.when(s % 2 == 0)
    def _():
        _attend(xcat1_ref, w3_ref, wproj_ref, bproj_ref, temp_ref,
                o_ref, **ak)
        _build_xcat(x_ref, xcat0_ref, cl, cr, rt, rb, **bk)

    @pl.when(s % 2 == 1)
    def _():
        _attend(xcat0_ref, w3_ref, wproj_ref, bproj_ref, temp_ref,
                o_ref, **ak)
        _build_xcat(x_ref, xcat1_ref, cl, cr, rt, rb, **bk)


def kernel(x, temperature, w_qkv, b_qkv, w_dw, b_dw, w_proj, b_proj):
    B, C, H, W = x.shape
    HW = H * W
    num_heads = temperature.shape[0]
    C3 = 3 * C

    wdw9 = w_dw.reshape(9, C3)
    w3 = jnp.einsum('ec,tc->cte', w_qkv, wdw9)
    w3 = w3.reshape(C3, 9 * C)
    wb9 = (b_qkv[None, :] * wdw9).T
    w3_full = jnp.concatenate(
        [w3, wb9, b_dw.reshape(C3, 1), jnp.zeros((C3, 6), w3.dtype)],
        axis=1).astype(jnp.bfloat16)

    bproj_c = b_proj.reshape(C, 1)
    temp_col = jnp.repeat(temperature.astype(jnp.float32),
                          C // num_heads).reshape(C, 1)

    body = functools.partial(_fused_attention_kernel,
                             H=H, W=W, num_heads=num_heads)
    K = 9 * C + 16
    out = pl.pallas_call(
        body,
        out_shape=jax.ShapeDtypeStruct((B, C, H, W), jnp.float32),
        grid=(B + 1,),
        in_specs=[
            pl.BlockSpec((1, C, H, W),
                         lambda b: (jnp.minimum(b, B - 1), 0, 0, 0)),
            pl.BlockSpec((C3, K), lambda b: (0, 0)),
            pl.BlockSpec((C, C), lambda b: (0, 0)),
            pl.BlockSpec((C, 1), lambda b: (0, 0)),
            pl.BlockSpec((C, 1), lambda b: (0, 0)),
        ],
        out_specs=pl.BlockSpec((1, C, H, W),
                               lambda b: (jnp.maximum(b - 1, 0), 0, 0, 0)),
        scratch_shapes=[pltpu.VMEM((K, HW), jnp.bfloat16),
                        pltpu.VMEM((K, HW), jnp.bfloat16)],
        compiler_params=pltpu.CompilerParams(
            dimension_semantics=("arbitrary",),
            vmem_limit_bytes=64 * 1024 * 1024,
        ),
    )(x, w3_full, w_proj, bproj_c, temp_col)
    return out
